# baseline (speedup 1.0000x reference)
import functools

import jax
import jax.numpy as jnp
from jax import lax
from jax.experimental import pallas as pl
from jax.experimental.pallas import tpu as pltpu
from jax.experimental.pallas import tpu_sc as plsc

B = 1024
TAU = 20
F = 10
V = 100000
D = 16
DM = 64
NL = 2
NT = 2
IN = DM + TAU * D * F
OUT = (TAU + 1) * DM
FD = F * D

NC = 2
NS = 16
NW = NC * NS
PPW = FD // NW
LANES = 16

TRX = IN // 8
TRE = FD // 8
TCB = B // 128


FPC = 2
SPLITS = F // FPC
PPC = FPC * D // NW


def _gather_core(f0, fut, tbl, x6, emb5, idx_v, row_v, buf0, buf1,
                 sem0, sem1, wid):
    bufs = (buf0, buf1)
    sems = (sem0, sem1)

    for k in range(PPC):
        p = PPC * wid + k
        fl = p // D
        P = f0 * D + p
        trb = P // 8
        s = P % 8
        if k == 0:
            pltpu.sync_copy(fut.at[fl], idx_v)
        else:
            @pl.when(fl != (p - 1) // D)
            def _():
                pltpu.sync_copy(fut.at[fl], idx_v)
        pltpu.sync_copy(tbl.at[p], row_v)

        def tt_body(tt, carry):
            for b in range(2):
                t = 2 * tt + b
                buf, sem = bufs[b], sems[b]

                @pl.when(tt > 0)
                def _():
                    pltpu.make_async_copy(
                        buf, x6.at[DM // 8 + TRE * t + trb, :, s], sem).wait()
                    pltpu.make_async_copy(
                        buf, emb5.at[t, trb, :, s], sem).wait()

                for c in range(B // LANES):
                    iv = idx_v[t, pl.ds(c * LANES, LANES)]
                    buf[c // 8, pl.ds((c % 8) * LANES, LANES)] = (
                        plsc.load_gather(row_v, [iv]))
                pltpu.async_copy(
                    buf, x6.at[DM // 8 + TRE * t + trb, :, s], sem)
                pltpu.async_copy(buf, emb5.at[t, trb, :, s], sem)
            return carry

        lax.fori_loop(0, TAU // 2, tt_body, 0)
        for b in range(2):
            t = TAU - 2 + b
            pltpu.make_async_copy(
                bufs[b], x6.at[DM // 8 + TRE * t + trb, :, s], sems[b]).wait()
            pltpu.make_async_copy(
                bufs[b], emb5.at[t, trb, :, s], sems[b]).wait()


def _make_gather(f0, with_hidden):
    if with_hidden:
        def body(fut, tbl, h5, x6, emb5, idx_v, row_v, buf0, buf1, hbuf,
                 sem0, sem1):
            wid = lax.axis_index("s") * NC + lax.axis_index("c")
            pltpu.sync_copy(
                h5.at[NL - 1, wid // 4, pl.ds((wid % 4) * 2, 2)], hbuf)
            pltpu.sync_copy(hbuf, x6.at[wid // 4, pl.ds((wid % 4) * 2, 2)])
            _gather_core(f0, fut, tbl, x6, emb5, idx_v, row_v, buf0, buf1,
                         sem0, sem1, wid)
        extra = [pltpu.VMEM((2, 8, 128), jnp.float32)]
    else:
        def body(fut, tbl, x6, emb5, idx_v, row_v, buf0, buf1, sem0, sem1):
            wid = lax.axis_index("s") * NC + lax.axis_index("c")
            _gather_core(f0, fut, tbl, x6, emb5, idx_v, row_v, buf0, buf1,
                         sem0, sem1, wid)
        extra = []
    return functools.partial(
        pl.kernel,
        mesh=plsc.VectorSubcoreMesh(
            core_axis_name="c", subcore_axis_name="s",
            num_cores=NC, num_subcores=NS),
        out_type=(),
        scratch_types=[
            pltpu.VMEM((TAU, B), jnp.int32),
            pltpu.VMEM((V,), jnp.float32),
            pltpu.VMEM((8, 128), jnp.float32),
            pltpu.VMEM((8, 128), jnp.float32),
        ] + extra + [
            pltpu.SemaphoreType.DMA,
            pltpu.SemaphoreType.DMA,
        ],
        compiler_params=pltpu.CompilerParams(
            use_tc_tiling_on_sc=False, needs_layout_passes=False,
            disable_bounds_checks=True),
        name=f"gather_f{f0}",
    )(body)


_gathers = [_make_gather(i * FPC, i == 0) for i in range(SPLITS)]


def _mm_body(x_ref, w_ref, b_ref, o_ref):
    xm = x_ref[...].reshape(TRX, 8, 128).reshape(IN, 128)
    acc = jnp.dot(w_ref[0], xm, preferred_element_type=jnp.float32)
    o_ref[0] = acc + b_ref[0]


_matmul = pl.pallas_call(
    _mm_body,
    grid=(NT, TCB),
    in_specs=[
        pl.BlockSpec((TRX, 1, 8, 128), lambda t, i: (0, i, 0, 0)),
        pl.BlockSpec((1, OUT, IN), lambda t, i: (t, 0, 0)),
        pl.BlockSpec((1, OUT, 1), lambda t, i: (t, 0, 0)),
    ],
    out_specs=pl.BlockSpec((1, OUT, 128), lambda t, i: (t, 0, i)),
    out_shape=jax.ShapeDtypeStruct((NT, OUT, B), jnp.float32),
)


def kernel(future, hidden, tables, W, b):
    fut = jnp.transpose(future.astype(jnp.int32), (2, 1, 0))
    tbl = jnp.transpose(tables, (0, 2, 1)).reshape(FD, V)
    h5 = (hidden.reshape(NL, B, 8, 8)
          .transpose(0, 2, 3, 1)
          .reshape(NL, 8, 8, 8, 128)
          .transpose(0, 1, 3, 2, 4))
    x6r = jax.new_ref(jnp.zeros((TRX, TCB, 8, 128), jnp.float32))
    emb5r = jax.new_ref(jnp.zeros((TAU, TRE, TCB, 8, 128), jnp.float32))
    for i in range(SPLITS):
        f0 = i * FPC
        fut_s = lax.slice_in_dim(fut, f0, f0 + FPC, axis=0)
        tbl_s = lax.slice_in_dim(tbl, f0 * D, (f0 + FPC) * D, axis=0)
        if i == 0:
            _gathers[i](fut_s, tbl_s, h5, x6r, emb5r)
        else:
            _gathers[i](fut_s, tbl_s, x6r, emb5r)
    x6 = x6r[...]
    emb5 = emb5r[...]
    emb_out = emb5.transpose(2, 4, 0, 1, 3).reshape(B, TAU, FD)
    gc_t = _matmul(x6, jnp.transpose(W, (0, 2, 1)), b.reshape(NT, OUT, 1))
    return emb_out, jnp.transpose(gc_t, (2, 0, 1))


# speedup vs baseline: 1.6280x; 1.6280x over previous
import functools

import jax
import jax.numpy as jnp
from jax import lax
from jax.experimental import pallas as pl
from jax.experimental.pallas import tpu as pltpu
from jax.experimental.pallas import tpu_sc as plsc

B = 1024
TAU = 20
F = 10
V = 100000
D = 16
DM = 64
NL = 2
NT = 2
IN = DM + TAU * D * F
OUT = (TAU + 1) * DM
FD = F * D

NC = 2
NS = 16
NW = NC * NS
PPW = FD // NW
LANES = 16

TRX = IN // 8
TRE = FD // 8
TCB = B // 128


def _gather_body(fut, tbl, h5, x6, emb5, idx_v, row_v, buf0, buf1, hbuf,
                 sem0, sem1):
    wid = lax.axis_index("s") * NC + lax.axis_index("c")

    pltpu.sync_copy(h5.at[NL - 1, wid // 4, pl.ds((wid % 4) * 2, 2)], hbuf)
    pltpu.sync_copy(hbuf, x6.at[wid // 4, pl.ds((wid % 4) * 2, 2)])

    bufs = (buf0, buf1)
    sems = (sem0, sem1)

    for k in range(PPW):
        p = PPW * wid + k
        f = p // D
        d = p % D
        trb = p // 8
        s = p % 8
        if k == 0:
            pltpu.sync_copy(fut.at[f], idx_v)
        else:
            @pl.when(f != (p - 1) // D)
            def _():
                pltpu.sync_copy(fut.at[f], idx_v)
        pltpu.sync_copy(tbl.at[f, d], row_v)

        def tt_body(tt, carry):
            for b in range(2):
                t = 2 * tt + b
                buf, sem = bufs[b], sems[b]

                @pl.when(tt > 0)
                def _():
                    pltpu.make_async_copy(
                        buf, x6.at[DM // 8 + TRE * t + trb, :, s], sem).wait()
                    pltpu.make_async_copy(
                        buf, emb5.at[t, trb, :, s], sem).wait()

                for c in range(B // LANES):
                    iv = idx_v[t, pl.ds(c * LANES, LANES)]
                    buf[c // 8, pl.ds((c % 8) * LANES, LANES)] = (
                        plsc.load_gather(row_v, [iv]))
                pltpu.async_copy(
                    buf, x6.at[DM // 8 + TRE * t + trb, :, s], sem)
                pltpu.async_copy(buf, emb5.at[t, trb, :, s], sem)
            return carry

        lax.fori_loop(0, TAU // 2, tt_body, 0)
        for b in range(2):
            t = TAU - 2 + b
            pltpu.make_async_copy(
                bufs[b], x6.at[DM // 8 + TRE * t + trb, :, s], sems[b]).wait()
            pltpu.make_async_copy(
                bufs[b], emb5.at[t, trb, :, s], sems[b]).wait()


_gather = functools.partial(
    pl.kernel,
    mesh=plsc.VectorSubcoreMesh(
        core_axis_name="c", subcore_axis_name="s",
        num_cores=NC, num_subcores=NS),
    out_type=(
        jax.ShapeDtypeStruct((TRX, TCB, 8, 128), jnp.float32),
        jax.ShapeDtypeStruct((TAU, TRE, TCB, 8, 128), jnp.float32),
    ),
    scratch_types=[
        pltpu.VMEM((TAU, B), jnp.int32),
        pltpu.VMEM((V,), jnp.float32),
        pltpu.VMEM((8, 128), jnp.float32),
        pltpu.VMEM((8, 128), jnp.float32),
        pltpu.VMEM((2, 8, 128), jnp.float32),
        pltpu.SemaphoreType.DMA,
        pltpu.SemaphoreType.DMA,
    ],
    compiler_params=pltpu.CompilerParams(
        use_tc_tiling_on_sc=True, needs_layout_passes=False,
        disable_bounds_checks=True),
)(_gather_body)


def _mm_body(x_ref, w_ref, b_ref, o_ref):
    xm = x_ref[...].reshape(TRX, 8, 128).reshape(IN, 128)
    acc = jnp.dot(w_ref[0], xm, preferred_element_type=jnp.float32)
    o_ref[0] = acc + b_ref[0]


_matmul = pl.pallas_call(
    _mm_body,
    grid=(NT, TCB),
    in_specs=[
        pl.BlockSpec((TRX, 1, 8, 128), lambda t, i: (0, i, 0, 0)),
        pl.BlockSpec((1, OUT, IN), lambda t, i: (t, 0, 0)),
        pl.BlockSpec((1, OUT, 1), lambda t, i: (t, 0, 0)),
    ],
    out_specs=pl.BlockSpec((1, OUT, 128), lambda t, i: (t, 0, i)),
    out_shape=jax.ShapeDtypeStruct((NT, OUT, B), jnp.float32),
)


def kernel(future, hidden, tables, W, b):
    fut = jnp.transpose(future.astype(jnp.int32), (2, 1, 0))
    tbl = jnp.transpose(tables, (0, 2, 1))
    h5 = (hidden.reshape(NL, B, 8, 8)
          .transpose(0, 2, 3, 1)
          .reshape(NL, 8, 8, 8, 128)
          .transpose(0, 1, 3, 2, 4))
    x6, emb5 = _gather(fut, tbl, h5)
    emb_out = emb5.transpose(2, 4, 0, 1, 3).reshape(B, TAU, FD)
    gc_t = _matmul(x6, jnp.transpose(W, (0, 2, 1)), b.reshape(NT, OUT, 1))
    return emb_out, jnp.transpose(gc_t, (2, 0, 1))
